# baseline (speedup 1.0000x reference)
import functools

import jax
import jax.numpy as jnp
from jax import lax
from jax.experimental import pallas as pl
from jax.experimental.pallas import tpu as pltpu
from jax.experimental.pallas import tpu_sc as plsc

VOCAB = 1000000
DIM = 64
BATCH = 16384
HIST = 50

NC = 2
NS = 16
NW = NC * NS
CHUNK = 128
N_CHUNKS_TOT = HIST * (BATCH // CHUNK)
PER_W = N_CHUNKS_TOT // NW
NB = 4
N_GROUPS = PER_W // NB
BT_N = BATCH // CHUNK


def _emb_body(idx_hbm, table_hbm, out_hbm, idx_v,
              r0, r1, r2, r3, t0, t1, g0, g1, g2, g3, w0, w1):
    rows = (r0, r1, r2, r3)
    gsems = (g0, g1, g2, g3)
    tiles = (t0, t1)
    wsems = (w0, w1)
    wid = lax.axis_index("s") * NC + lax.axis_index("c")
    cbase = wid * PER_W

    pltpu.sync_copy(idx_hbm.at[wid], idx_v)

    lanes = lax.iota(jnp.int32, 16)
    lanevecs = [lanes + lg * 16 for lg in range(8)]

    for k in range(NB):
        pltpu.async_copy(table_hbm.at[idx_v.at[k]], rows[k], gsems[k])

    def group(gi, carry):
        for j in range(NB):
            k = gi * NB + j
            c = cbase + k
            h = lax.shift_right_logical(c, 7)
            bt = lax.bitwise_and(c, BT_N - 1)
            rb = rows[j]
            tb = tiles[j & 1]
            wsem = wsems[j & 1]

            pltpu.make_async_copy(
                table_hbm.at[idx_v.at[0]], rb, gsems[j]).wait()

            @pl.when(k >= 2)
            def _drain():
                for _ in range(8):
                    pltpu.make_async_copy(
                        tb.at[pl.ds(0, 1024)], out_hbm.at[0, 0, 0],
                        wsem).wait()

            def dd_step(dd, carry2):
                ddv = jnp.full((16,), 0, jnp.int32) + dd
                for lg in range(8):
                    vec = plsc.load_gather(rb, [lanevecs[lg], ddv])
                    tb[pl.ds(dd * CHUNK + lg * 16, 16)] = vec
                return carry2

            lax.fori_loop(0, DIM, dd_step, 0)

            @pl.when(k + NB < PER_W)
            def _prefetch():
                pltpu.async_copy(
                    table_hbm.at[idx_v.at[k + NB]], rb, gsems[j])

            for tr in range(8):
                pltpu.async_copy(
                    tb.at[pl.ds(tr * 1024, 1024)],
                    out_hbm.at[h, tr, bt], wsem)

        return carry

    lax.fori_loop(0, N_GROUPS, group, 0)

    for wsem in wsems:
        for _ in range(8):
            pltpu.make_async_copy(
                t0.at[pl.ds(0, 1024)], out_hbm.at[0, 0, 0], wsem).wait()


_emb = functools.partial(
    pl.kernel,
    out_type=jax.ShapeDtypeStruct((HIST, 8, BT_N, 8 * CHUNK), jnp.float32),
    mesh=plsc.VectorSubcoreMesh(core_axis_name="c", subcore_axis_name="s"),
    scratch_types=[
        pltpu.VMEM((PER_W, CHUNK), jnp.int32),
    ] + [pltpu.VMEM((CHUNK, DIM), jnp.float32)] * NB
      + [pltpu.VMEM((DIM * CHUNK,), jnp.float32)] * 2
      + [pltpu.SemaphoreType.DMA] * (NB + 2),
    compiler_params=pltpu.CompilerParams(
        use_tc_tiling_on_sc=False, needs_layout_passes=False),
)(_emb_body)


@jax.jit
def kernel(x, table):
    xq = x.T.astype(jnp.int32).reshape(NW, PER_W, CHUNK)
    out4 = _emb(xq, table)
    out5 = out4.reshape(HIST, 8, BT_N, 8, CHUNK)
    return out5.transpose(2, 4, 0, 1, 3).reshape(BATCH, HIST, DIM)


# speedup vs baseline: 1.1630x; 1.1630x over previous
import functools

import jax
import jax.numpy as jnp
from jax import lax
from jax.experimental import pallas as pl
from jax.experimental.pallas import tpu as pltpu
from jax.experimental.pallas import tpu_sc as plsc

VOCAB = 1000000
DIM = 64
BATCH = 16384
HIST = 50

NC = 2
NS = 16
NW = NC * NS
CHUNK = 128
N_CHUNKS_TOT = HIST * (BATCH // CHUNK)
PER_W = N_CHUNKS_TOT // NW
NB = 4
N_GROUPS = PER_W // NB
BT_N = BATCH // CHUNK


def _emb_body(idx_hbm, table_hbm, out_hbm, idx_v,
              r0, r1, r2, r3, t0, t1, g0, g1, g2, g3, w0, w1):
    rows = (r0, r1, r2, r3)
    gsems = (g0, g1, g2, g3)
    tiles = (t0, t1)
    wsems = (w0, w1)
    wid = lax.axis_index("s") * NC + lax.axis_index("c")
    cbase = wid * PER_W

    pltpu.sync_copy(idx_hbm.at[wid], idx_v)

    lanes = lax.iota(jnp.int32, 16)
    lanevecs = [lanes + lg * 16 for lg in range(8)]

    for k in range(NB):
        pltpu.async_copy(table_hbm.at[idx_v.at[k]], rows[k], gsems[k])

    def group(gi, carry):
        for j in range(NB):
            k = gi * NB + j
            c = cbase + k
            h = lax.shift_right_logical(c, 7)
            bt = lax.bitwise_and(c, BT_N - 1)
            rb = rows[j]
            tb = tiles[j & 1]
            wsem = wsems[j & 1]

            pltpu.make_async_copy(
                table_hbm.at[idx_v.at[0]], rb, gsems[j]).wait()

            @pl.when(k >= 2)
            def _drain():
                for _ in range(8):
                    pltpu.make_async_copy(
                        tb.at[pl.ds(0, 1024)], out_hbm.at[0, 0, 0],
                        wsem).wait()

            def dd_step(di, carry2):
                dd0 = di * 4
                vecs = []
                for u in range(4):
                    ddv = jnp.full((16,), 0, jnp.int32) + (dd0 + u)
                    for lg in range(8):
                        vecs.append(plsc.load_gather(rb, [lanevecs[lg], ddv]))
                for u in range(4):
                    for lg in range(8):
                        tb[pl.ds((dd0 + u) * CHUNK + lg * 16, 16)] = (
                            vecs[u * 8 + lg])
                return carry2

            lax.fori_loop(0, DIM // 4, dd_step, 0)

            @pl.when(k + NB < PER_W)
            def _prefetch():
                pltpu.async_copy(
                    table_hbm.at[idx_v.at[k + NB]], rb, gsems[j])

            for tr in range(8):
                pltpu.async_copy(
                    tb.at[pl.ds(tr * 1024, 1024)],
                    out_hbm.at[h, tr, bt], wsem)

        return carry

    lax.fori_loop(0, N_GROUPS, group, 0)

    for wsem in wsems:
        for _ in range(8):
            pltpu.make_async_copy(
                t0.at[pl.ds(0, 1024)], out_hbm.at[0, 0, 0], wsem).wait()


_emb = functools.partial(
    pl.kernel,
    out_type=jax.ShapeDtypeStruct((HIST, 8, BT_N, 8 * CHUNK), jnp.float32),
    mesh=plsc.VectorSubcoreMesh(core_axis_name="c", subcore_axis_name="s"),
    scratch_types=[
        pltpu.VMEM((PER_W, CHUNK), jnp.int32),
    ] + [pltpu.VMEM((CHUNK, DIM), jnp.float32)] * NB
      + [pltpu.VMEM((DIM * CHUNK,), jnp.float32)] * 2
      + [pltpu.SemaphoreType.DMA] * (NB + 2),
    compiler_params=pltpu.CompilerParams(
        use_tc_tiling_on_sc=False, needs_layout_passes=False),
)(_emb_body)


@jax.jit
def kernel(x, table):
    xq = x.T.astype(jnp.int32).reshape(NW, PER_W, CHUNK)
    out4 = _emb(xq, table)
    out5 = out4.reshape(HIST, 8, BT_N, 8, CHUNK)
    return out5.transpose(2, 4, 0, 1, 3).reshape(BATCH, HIST, DIM)


# speedup vs baseline: 1.8279x; 1.5717x over previous
import functools

import jax
import jax.numpy as jnp
from jax import lax
from jax.experimental import pallas as pl
from jax.experimental.pallas import tpu as pltpu
from jax.experimental.pallas import tpu_sc as plsc

VOCAB = 1000000
DIM = 64
BATCH = 16384
HIST = 50

NC = 2
NS = 16
NW = NC * NS
CHUNK = 128
N_CHUNKS_TOT = HIST * (BATCH // CHUNK)
PER_W = N_CHUNKS_TOT // NW
NB = 4
N_GROUPS = PER_W // NB
BT_N = BATCH // CHUNK


def _emb_body(idx_hbm, table_hbm, out_hbm, idx_v,
              r0, r1, r2, r3, t0, t1, g0, g1, g2, g3, w0, w1):
    rows = (r0, r1, r2, r3)
    gsems = (g0, g1, g2, g3)
    tiles = (t0, t1)
    wsems = (w0, w1)
    wid = lax.axis_index("s") * NC + lax.axis_index("c")
    cbase = wid * PER_W

    pltpu.sync_copy(idx_hbm.at[wid], idx_v)

    lanes = lax.iota(jnp.int32, 16)
    lanevecs = [lanes + lg * 16 for lg in range(8)]

    for k in range(NB):
        pltpu.async_copy(table_hbm.at[idx_v.at[k]], rows[k], gsems[k])

    def group(gi, carry):
        for j in range(NB):
            k = gi * NB + j
            c = cbase + k
            h = lax.shift_right_logical(c, 7)
            bt = lax.bitwise_and(c, BT_N - 1)
            rb = rows[j]
            tb = tiles[j & 1]
            wsem = wsems[j & 1]

            pltpu.make_async_copy(
                table_hbm.at[idx_v.at[0]], rb, gsems[j]).wait()

            @pl.when(k >= 2)
            def _drain():
                for _ in range(8):
                    pltpu.make_async_copy(
                        tb.at[pl.ds(0, 8), pl.ds(0, CHUNK)],
                        out_hbm.at[0, 0, 0], wsem).wait()

            def l_step(li, carry2):
                l0 = li * 2
                for u in range(2):
                    lv = jnp.full((16,), 0, jnp.int32) + (l0 + u)
                    for g in range(4):
                        vec = rb[l0 + u, pl.ds(g * 16, 16)]
                        plsc.store_scatter(
                            tb, [lanevecs[g], lv], vec)
                return carry2

            lax.fori_loop(0, CHUNK // 2, l_step, 0)

            @pl.when(k + NB < PER_W)
            def _prefetch():
                pltpu.async_copy(
                    table_hbm.at[idx_v.at[k + NB]],
                    rb.at[:, pl.ds(0, DIM)], gsems[j])

            for tr in range(8):
                pltpu.async_copy(
                    tb.at[pl.ds(tr * 8, 8), pl.ds(0, CHUNK)],
                    out_hbm.at[h, tr, bt], wsem)

        return carry

    lax.fori_loop(0, N_GROUPS, group, 0)

    for wsem in wsems:
        for _ in range(8):
            pltpu.make_async_copy(
                t0.at[pl.ds(0, 8), pl.ds(0, CHUNK)],
                out_hbm.at[0, 0, 0], wsem).wait()


_emb = functools.partial(
    pl.kernel,
    out_type=jax.ShapeDtypeStruct((HIST, 8, BT_N, 8, CHUNK), jnp.float32),
    mesh=plsc.VectorSubcoreMesh(core_axis_name="c", subcore_axis_name="s"),
    scratch_types=[
        pltpu.VMEM((PER_W, CHUNK), jnp.int32),
    ] + [pltpu.VMEM((CHUNK, DIM), jnp.float32)] * NB
      + [pltpu.VMEM((DIM, CHUNK + 1), jnp.float32)] * 2
      + [pltpu.SemaphoreType.DMA] * (NB + 2),
    compiler_params=pltpu.CompilerParams(
        use_tc_tiling_on_sc=False, needs_layout_passes=False),
)(_emb_body)


@jax.jit
def kernel(x, table):
    xq = x.T.astype(jnp.int32).reshape(NW, PER_W, CHUNK)
    out5 = _emb(xq, table)
    return out5.transpose(2, 4, 0, 1, 3).reshape(BATCH, HIST, DIM)


# speedup vs baseline: 1.9483x; 1.0659x over previous
import functools

import jax
import jax.numpy as jnp
from jax import lax
from jax.experimental import pallas as pl
from jax.experimental.pallas import tpu as pltpu
from jax.experimental.pallas import tpu_sc as plsc

VOCAB = 1000000
DIM = 64
BATCH = 16384
HIST = 50

NC = 2
NS = 16
NW = NC * NS
CHUNK = 128
N_CHUNKS_TOT = HIST * (BATCH // CHUNK)
PER_W = N_CHUNKS_TOT // NW
NB = 4
N_GROUPS = PER_W // NB
BT_N = BATCH // CHUNK


def _emb_body(idx_hbm, table_hbm, out_hbm, idx_v,
              r0, r1, r2, r3, t0, t1, g0, g1, g2, g3, w0, w1):
    rows = (r0, r1, r2, r3)
    gsems = (g0, g1, g2, g3)
    tiles = (t0, t1)
    wsems = (w0, w1)
    wid = lax.axis_index("s") * NC + lax.axis_index("c")
    cbase = wid * PER_W

    pltpu.sync_copy(idx_hbm.at[wid], idx_v)

    lanes = lax.iota(jnp.int32, 16)
    lanevecs = [lanes + lg * 16 for lg in range(8)]

    for k in range(NB):
        pltpu.async_copy(table_hbm.at[idx_v.at[k]], rows[k], gsems[k])

    def group(gi, carry):
        for j in range(NB):
            k = gi * NB + j
            c = cbase + k
            h = lax.shift_right_logical(c, 7)
            bt = lax.bitwise_and(c, BT_N - 1)
            rb = rows[j]
            tb = tiles[j & 1]
            wsem = wsems[j & 1]

            pltpu.make_async_copy(
                table_hbm.at[idx_v.at[0]], rb, gsems[j]).wait()

            @pl.when(k >= 2)
            def _drain():
                for _ in range(8):
                    pltpu.make_async_copy(
                        tb.at[pl.ds(0, 8), pl.ds(0, CHUNK)],
                        out_hbm.at[0, 0, 0], wsem).wait()

            def l_step(li, carry2):
                l0 = li * 2
                for u in range(2):
                    lv = jnp.full((16,), 0, jnp.int32) + (l0 + u)
                    for g in range(4):
                        vec = rb[l0 + u, pl.ds(g * 16, 16)]
                        plsc.store_scatter(
                            tb, [lanevecs[g], lv], vec)
                return carry2

            lax.fori_loop(0, CHUNK // 2, l_step, 0)

            @pl.when(k + NB < PER_W)
            def _prefetch():
                pltpu.async_copy(
                    table_hbm.at[idx_v.at[k + NB]], rb, gsems[j])

            for tr in range(8):
                pltpu.async_copy(
                    tb.at[pl.ds(tr * 8, 8), pl.ds(0, CHUNK)],
                    out_hbm.at[h, tr, bt], wsem)

        return carry

    lax.fori_loop(0, N_GROUPS, group, 0)

    for wsem in wsems:
        for _ in range(8):
            pltpu.make_async_copy(
                t0.at[pl.ds(0, 8), pl.ds(0, CHUNK)],
                out_hbm.at[0, 0, 0], wsem).wait()


_emb = functools.partial(
    pl.kernel,
    out_type=jax.ShapeDtypeStruct((HIST, 8, BT_N, 8, CHUNK), jnp.float32),
    mesh=plsc.VectorSubcoreMesh(core_axis_name="c", subcore_axis_name="s"),
    scratch_types=[
        pltpu.VMEM((PER_W, CHUNK), jnp.int32),
    ] + [pltpu.VMEM((CHUNK, 2 * DIM), jnp.float32)] * NB
      + [pltpu.VMEM((DIM, CHUNK + 1), jnp.float32)] * 2
      + [pltpu.SemaphoreType.DMA] * (NB + 2),
    compiler_params=pltpu.CompilerParams(
        use_tc_tiling_on_sc=False, needs_layout_passes=False),
)(_emb_body)


@jax.jit
def kernel(x, table):
    xq = x.T.astype(jnp.int32).reshape(NW, PER_W, CHUNK)
    tab128 = jnp.concatenate(
        [table, jnp.zeros((VOCAB, DIM), jnp.float32)], axis=1)
    out5 = _emb(xq, tab128)
    return out5.transpose(2, 4, 0, 1, 3).reshape(BATCH, HIST, DIM)


# speedup vs baseline: 1.9630x; 1.0075x over previous
import functools

import jax
import jax.numpy as jnp
from jax import lax
from jax.experimental import pallas as pl
from jax.experimental.pallas import tpu as pltpu
from jax.experimental.pallas import tpu_sc as plsc

VOCAB = 1000000
DIM = 64
BATCH = 16384
HIST = 50

NC = 2
NS = 16
NW = NC * NS
CHUNK = 128
N_CHUNKS_TOT = HIST * (BATCH // CHUNK)
PER_W = N_CHUNKS_TOT // NW
NB = 4
N_GROUPS = PER_W // NB
BT_N = BATCH // CHUNK


def _emb_body(idx_hbm, table_hbm, out_hbm, idx_v,
              r0, r1, r2, r3, t0, t1, g0, g1, g2, g3, w0, w1):
    rows = (r0, r1, r2, r3)
    gsems = (g0, g1, g2, g3)
    tiles = (t0, t1)
    wsems = (w0, w1)
    wid = lax.axis_index("s") * NC + lax.axis_index("c")
    cbase = wid * PER_W

    pltpu.sync_copy(idx_hbm.at[wid], idx_v)

    lanes = lax.iota(jnp.int32, 16)
    lanevecs = [lanes + lg * 16 for lg in range(8)]

    for k in range(NB):
        pltpu.async_copy(table_hbm.at[idx_v.at[k]], rows[k], gsems[k])

    def group(gi, carry):
        for j in range(NB):
            k = gi * NB + j
            c = cbase + k
            h = lax.shift_right_logical(c, 7)
            bt = lax.bitwise_and(c, BT_N - 1)
            rb = rows[j]
            tb = tiles[j & 1]
            wsem = wsems[j & 1]

            pltpu.make_async_copy(
                table_hbm.at[idx_v.at[0]], rb, gsems[j]).wait()

            @pl.when(k >= 2)
            def _drain():
                for _ in range(8):
                    pltpu.make_async_copy(
                        tb.at[pl.ds(0, 8), pl.ds(0, CHUNK)],
                        out_hbm.at[0, 0, 0], wsem).wait()

            def l_step(li, carry2):
                l0 = li * 4
                for u in range(4):
                    lv = jnp.full((16,), 0, jnp.int32) + (l0 + u)
                    for g in range(4):
                        vec = rb[l0 + u, pl.ds(g * 16, 16)]
                        plsc.store_scatter(
                            tb, [lanevecs[g], lv], vec)
                return carry2

            lax.fori_loop(0, CHUNK // 4, l_step, 0)

            @pl.when(k + NB < PER_W)
            def _prefetch():
                pltpu.async_copy(
                    table_hbm.at[idx_v.at[k + NB]], rb, gsems[j])

            for tr in range(8):
                pltpu.async_copy(
                    tb.at[pl.ds(tr * 8, 8), pl.ds(0, CHUNK)],
                    out_hbm.at[h, tr, bt], wsem)

        return carry

    lax.fori_loop(0, N_GROUPS, group, 0)

    for wsem in wsems:
        for _ in range(8):
            pltpu.make_async_copy(
                t0.at[pl.ds(0, 8), pl.ds(0, CHUNK)],
                out_hbm.at[0, 0, 0], wsem).wait()


_emb = functools.partial(
    pl.kernel,
    out_type=jax.ShapeDtypeStruct((HIST, 8, BT_N, 8, CHUNK), jnp.float32),
    mesh=plsc.VectorSubcoreMesh(core_axis_name="c", subcore_axis_name="s"),
    scratch_types=[
        pltpu.VMEM((PER_W, CHUNK), jnp.int32),
    ] + [pltpu.VMEM((CHUNK, 2 * DIM), jnp.float32)] * NB
      + [pltpu.VMEM((DIM, CHUNK + 1), jnp.float32)] * 2
      + [pltpu.SemaphoreType.DMA] * (NB + 2),
    compiler_params=pltpu.CompilerParams(
        use_tc_tiling_on_sc=False, needs_layout_passes=False),
)(_emb_body)


@jax.jit
def kernel(x, table):
    xq = x.T.astype(jnp.int32).reshape(NW, PER_W, CHUNK)
    tab128 = jnp.concatenate(
        [table, jnp.zeros((VOCAB, DIM), jnp.float32)], axis=1)
    out5 = _emb(xq, tab128)
    return out5.transpose(2, 4, 0, 1, 3).reshape(BATCH, HIST, DIM)


# speedup vs baseline: 1.9661x; 1.0016x over previous
import functools

import jax
import jax.numpy as jnp
from jax import lax
from jax.experimental import pallas as pl
from jax.experimental.pallas import tpu as pltpu
from jax.experimental.pallas import tpu_sc as plsc

VOCAB = 1000000
DIM = 64
BATCH = 16384
HIST = 50

NC = 2
NS = 16
NW = NC * NS
CHUNK = 128
N_CHUNKS_TOT = HIST * (BATCH // CHUNK)
PER_W = N_CHUNKS_TOT // NW
NB = 4
N_GROUPS = PER_W // NB
BT_N = BATCH // CHUNK


def _emb_body(idx_hbm, table_hbm, out_hbm, idx_v,
              r0, r1, r2, r3, t0, t1, g0, g1, g2, g3, w0, w1):
    rows = (r0, r1, r2, r3)
    gsems = (g0, g1, g2, g3)
    tiles = (t0, t1)
    wsems = (w0, w1)
    wid = lax.axis_index("s") * NC + lax.axis_index("c")
    cbase = wid * PER_W

    pltpu.sync_copy(idx_hbm.at[wid], idx_v)

    lanes = lax.iota(jnp.int32, 16)
    lanevecs = [lanes + lg * 16 for lg in range(8)]

    for k in range(NB):
        pltpu.async_copy(table_hbm.at[idx_v.at[k]], rows[k], gsems[k])

    def group(gi, carry):
        for j in range(NB):
            k = gi * NB + j
            c = cbase + k
            h = lax.shift_right_logical(c, 7)
            bt = lax.bitwise_and(c, BT_N - 1)
            rb = rows[j]
            tb = tiles[j & 1]
            wsem = wsems[j & 1]

            pltpu.make_async_copy(
                table_hbm.at[idx_v.at[0]], rb, gsems[j]).wait()

            @pl.when(k >= 2)
            def _drain():
                for _ in range(8):
                    pltpu.make_async_copy(
                        tb.at[pl.ds(0, 8), pl.ds(0, CHUNK)],
                        out_hbm.at[0, 0, 0], wsem).wait()

            def l_step(li, carry2):
                l0 = li * 4
                for u in range(4):
                    lv = jnp.full((16,), 0, jnp.int32) + (l0 + u)
                    for g in range(4):
                        vec = rb[l0 + u, pl.ds(g * 16, 16)]
                        plsc.store_scatter(
                            tb, [lanevecs[g], lv], vec)
                return carry2

            lax.fori_loop(0, CHUNK // 4, l_step, 0)

            @pl.when(k + NB < PER_W)
            def _prefetch():
                pltpu.async_copy(
                    table_hbm.at[idx_v.at[k + NB]], rb, gsems[j])

            for tr in range(8):
                pltpu.async_copy(
                    tb.at[pl.ds(tr * 8, 8), pl.ds(0, CHUNK)],
                    out_hbm.at[h, tr, bt], wsem)

        return carry

    lax.fori_loop(0, N_GROUPS, group, 0)

    for wsem in wsems:
        for _ in range(8):
            pltpu.make_async_copy(
                t0.at[pl.ds(0, 8), pl.ds(0, CHUNK)],
                out_hbm.at[0, 0, 0], wsem).wait()


_emb = functools.partial(
    pl.kernel,
    out_type=jax.ShapeDtypeStruct((HIST, 8, BT_N, 8, CHUNK), jnp.float32),
    mesh=plsc.VectorSubcoreMesh(core_axis_name="c", subcore_axis_name="s"),
    scratch_types=[
        pltpu.VMEM((PER_W, CHUNK), jnp.int32),
    ] + [pltpu.VMEM((CHUNK, DIM), jnp.float32)] * NB
      + [pltpu.VMEM((DIM, CHUNK + 1), jnp.float32)] * 2
      + [pltpu.SemaphoreType.DMA] * (NB + 2),
    compiler_params=pltpu.CompilerParams(
        use_tc_tiling_on_sc=False, needs_layout_passes=False),
)(_emb_body)


@jax.jit
def kernel(x, table):
    xq = (x.T.astype(jnp.int32) * 2).reshape(NW, PER_W, CHUNK)
    tab128 = jnp.concatenate(
        [table, jnp.zeros((VOCAB, DIM), jnp.float32)], axis=1)
    tab2m = tab128.reshape(2 * VOCAB, DIM)
    out5 = _emb(xq, tab2m)
    return out5.transpose(2, 4, 0, 1, 3).reshape(BATCH, HIST, DIM)
